# baseline (speedup 1.0000x reference)
import functools

import jax
import jax.numpy as jnp
from jax import lax
from jax.experimental import pallas as pl
from jax.experimental.pallas import tpu as pltpu
from jax.experimental.pallas import tpu_sc as plsc

M = 8
K = 8192
D = 256

NC = 2
NS = 16
NW = NC * NS

CB = 16
IDX = CB * M
LANES = 16
NBUF = 2


def _body(b_per_w, idx_hbm, table_hbm, out_hbm,
          idx_v, rows_bufs, out_bufs, sgs, sos):
    wid = lax.axis_index("s") * NC + lax.axis_index("c")
    base = wid * b_per_w
    n_chunks = b_per_w // CB
    n_groups = n_chunks // NBUF

    pltpu.sync_copy(idx_hbm.at[pl.ds(base * M, IDX)],
                    idx_v.at[pl.ds(0, IDX)])

    def fire_gather(c, p):
        pltpu.async_copy(
            table_hbm.at[idx_v.at[pl.ds(c * IDX, IDX)]], rows_bufs[p], sgs[p])

    def wait_gather(p):
        pltpu.make_async_copy(
            table_hbm.at[idx_v.at[pl.ds(0, IDX)]], rows_bufs[p], sgs[p]).wait()

    def out_slice(c):
        return out_hbm.at[pl.ds(base + c * CB, CB)]

    def wait_out(p):
        pltpu.make_async_copy(out_bufs[p], out_slice(0), sos[p]).wait()

    def compute(p):
        rv, ov = rows_bufs[p], out_bufs[p]

        @plsc.parallel_loop(0, CB, step=1, unroll=2)
        def row_body(r):
            for d in range(D // LANES):
                sl = pl.ds(d * LANES, LANES)
                s0 = rv[r * M + 0, sl] + rv[r * M + 1, sl]
                s1 = rv[r * M + 2, sl] + rv[r * M + 3, sl]
                s2 = rv[r * M + 4, sl] + rv[r * M + 5, sl]
                s3 = rv[r * M + 6, sl] + rv[r * M + 7, sl]
                ov[r, sl] = (s0 + s1) + (s2 + s3)

    fire_gather(0, 0)
    pltpu.sync_copy(idx_hbm.at[pl.ds(base * M + IDX, b_per_w * M - IDX)],
                    idx_v.at[pl.ds(IDX, b_per_w * M - IDX)])

    for c in range(NBUF):
        fire_gather(c + 1, (c + 1) % NBUF)
        wait_gather(c)
        compute(c)
        pltpu.async_copy(out_bufs[c], out_slice(c), sos[c])

    def group_body(i, carry):
        for j in range(NBUF):
            c = NBUF * i + j
            fire_gather(c + 1, (j + 1) % NBUF)
            wait_gather(j)
            wait_out(j)
            compute(j)
            pltpu.async_copy(out_bufs[j], out_slice(c), sos[j])
        return carry

    lax.fori_loop(1, n_groups - 1, group_body, 0, unroll=False)

    for c in range(n_chunks - NBUF, n_chunks):
        p = c % NBUF
        if c + 1 < n_chunks:
            fire_gather(c + 1, (p + 1) % NBUF)
        wait_gather(p)
        wait_out(p)
        compute(p)
        pltpu.async_copy(out_bufs[p], out_slice(c), sos[p])
    for p in range(NBUF):
        wait_out(p)


@jax.jit
def kernel(pickedIndices, codebook):
    B = pickedIndices.shape[0]
    b_per_w = B // NW
    table = codebook.reshape(M * K, D)
    idx_flat = (pickedIndices + jnp.arange(M, dtype=jnp.int32)[None, :] * K
                ).reshape(B * M)

    mesh = plsc.VectorSubcoreMesh(core_axis_name="c", subcore_axis_name="s")
    run = pl.kernel(
        functools.partial(_body, b_per_w),
        out_type=jax.ShapeDtypeStruct((B, D), jnp.float32),
        mesh=mesh,
        scratch_types=[
            pltpu.VMEM((b_per_w * M,), jnp.int32),
            [pltpu.VMEM((IDX, D), jnp.float32) for _ in range(NBUF)],
            [pltpu.VMEM((CB, D), jnp.float32) for _ in range(NBUF)],
            [pltpu.SemaphoreType.DMA for _ in range(NBUF)],
            [pltpu.SemaphoreType.DMA for _ in range(NBUF)],
        ],
    )
    return run(idx_flat, table)


# speedup vs baseline: 1.0204x; 1.0204x over previous
import functools

import jax
import jax.numpy as jnp
from jax import lax
from jax.experimental import pallas as pl
from jax.experimental.pallas import tpu as pltpu
from jax.experimental.pallas import tpu_sc as plsc

M = 8
K = 8192
D = 256

NC = 2
NS = 16
NW = NC * NS

CB = 16
IDX = CB * M
LANES = 16
NBUF = 2


def _body(b_per_w, idx_hbm, table_hbm, out_hbm,
          idx_v, rows_bufs, out_bufs, sgs, sos):
    wid = lax.axis_index("s") * NC + lax.axis_index("c")
    base = wid * b_per_w
    n_chunks = b_per_w // CB
    n_groups = n_chunks // NBUF

    pltpu.sync_copy(idx_hbm.at[pl.ds(base * M, IDX)],
                    idx_v.at[pl.ds(0, IDX)])

    def fire_gather(c, p):
        pltpu.async_copy(
            table_hbm.at[idx_v.at[pl.ds(c * IDX, IDX)]], rows_bufs[p], sgs[p])

    def wait_gather(p):
        pltpu.make_async_copy(
            table_hbm.at[idx_v.at[pl.ds(0, IDX)]], rows_bufs[p], sgs[p]).wait()

    def out_slice(c):
        return out_hbm.at[pl.ds(base + c * CB, CB)]

    def wait_out(p):
        pltpu.make_async_copy(out_bufs[p], out_slice(0), sos[p]).wait()

    def compute(p):
        rv, ov = rows_bufs[p], out_bufs[p]

        @plsc.parallel_loop(0, CB, step=1, unroll=2)
        def row_body(r):
            for d in range(D // LANES):
                sl = pl.ds(d * LANES, LANES)
                s0 = rv[r * M + 0, sl] + rv[r * M + 1, sl]
                s1 = rv[r * M + 2, sl] + rv[r * M + 3, sl]
                s2 = rv[r * M + 4, sl] + rv[r * M + 5, sl]
                s3 = rv[r * M + 6, sl] + rv[r * M + 7, sl]
                ov[r, sl] = (s0 + s1) + (s2 + s3)

    fire_gather(0, 0)
    pltpu.sync_copy(idx_hbm.at[pl.ds(base * M + IDX, b_per_w * M - IDX)],
                    idx_v.at[pl.ds(IDX, b_per_w * M - IDX)])

    def group_body(i, carry):
        for j in range(NBUF):
            c = NBUF * i + j
            pf = (j + NBUF - 1) % NBUF

            @pl.when(c + NBUF - 1 < n_chunks)
            def _():
                fire_gather(c + NBUF - 1, pf)

            wait_gather(j)

            @pl.when(i > 0)
            def _():
                wait_out(j)

            compute(j)
            pltpu.async_copy(out_bufs[j], out_slice(c), sos[j])
        return carry

    lax.fori_loop(0, n_groups, group_body, 0, unroll=False)
    for c in range(n_groups * NBUF, n_chunks):
        p = c % NBUF
        wait_gather(p)
        wait_out(p)
        compute(p)
        pltpu.async_copy(out_bufs[p], out_slice(c), sos[p])
    for p in range(NBUF):
        wait_out(p)


@jax.jit
def kernel(pickedIndices, codebook):
    B = pickedIndices.shape[0]
    b_per_w = B // NW
    table = codebook.reshape(M * K, D)
    idx_flat = (pickedIndices + jnp.arange(M, dtype=jnp.int32)[None, :] * K
                ).reshape(B * M)

    mesh = plsc.VectorSubcoreMesh(core_axis_name="c", subcore_axis_name="s")
    run = pl.kernel(
        functools.partial(_body, b_per_w),
        out_type=jax.ShapeDtypeStruct((B, D), jnp.float32),
        mesh=mesh,
        scratch_types=[
            pltpu.VMEM((b_per_w * M,), jnp.int32),
            [pltpu.VMEM((IDX, D), jnp.float32) for _ in range(NBUF)],
            [pltpu.VMEM((CB, D), jnp.float32) for _ in range(NBUF)],
            [pltpu.SemaphoreType.DMA for _ in range(NBUF)],
            [pltpu.SemaphoreType.DMA for _ in range(NBUF)],
        ],
    )
    return run(idx_flat, table)
